# Initial kernel scaffold; baseline (speedup 1.0000x reference)
import jax
import jax.numpy as jnp
from jax.experimental import pallas as pl


def kernel(x, target):
    raise NotImplementedError("write your pallas kernel here")


# speedup vs baseline: 4.4175x; 4.4175x over previous
import functools

import jax
import jax.numpy as jnp
from jax.experimental import pallas as pl
from jax.experimental.pallas import tpu as pltpu

_SIZE = 32000
_N = 4096
_SMOOTHING = 0.1
_CONF = 1.0 - _SMOOTHING
_S = _SMOOTHING / (_SIZE - 2)

_BR = 16


def _body(x_ref, tgt_ref, out_ref, acc_ref):
    i = pl.program_id(0)

    @pl.when(i == 0)
    def _init():
        acc_ref[0] = 0.0

    xb = x_ref[...]
    m = jnp.max(xb, axis=1)
    se = jnp.sum(jnp.exp(xb - m[:, None]), axis=1)
    lse = m + jnp.log(se)
    sumx = jnp.sum(xb, axis=1)
    x0 = xb[:, 0]

    tgt = tgt_ref[0, 0, :]
    col = jax.lax.broadcasted_iota(jnp.int32, (_BR, _SIZE), 1)
    xt = jnp.sum(jnp.where(col == tgt[:, None], xb, 0.0), axis=1)

    c0 = _CONF * jnp.log(_CONF) + (_SIZE - 2) * _S * jnp.log(_S)
    kl = jnp.where(tgt != 0, c0 + lse - _CONF * xt - _S * (sumx - x0 - xt), 0.0)
    acc_ref[0] += jnp.sum(kl)

    @pl.when(i == pl.num_programs(0) - 1)
    def _fin():
        out_ref[0] = acc_ref[0]


@functools.partial(jax.jit, static_argnames=())
def kernel(x, target):
    n, size = x.shape
    grid = n // _BR
    out = pl.pallas_call(
        _body,
        grid=(grid,),
        in_specs=[
            pl.BlockSpec((_BR, size), lambda i: (i, 0)),
            pl.BlockSpec((1, 1, _BR), lambda i: (i, 0, 0)),
        ],
        out_specs=pl.BlockSpec(memory_space=pltpu.SMEM),
        out_shape=jax.ShapeDtypeStruct((1,), jnp.float32),
        scratch_shapes=[pltpu.SMEM((1,), jnp.float32)],
    )(x, target.reshape(grid, 1, _BR))
    return out[0]
